# baseline (speedup 1.0000x reference)
import functools

import jax
import jax.numpy as jnp
from jax import lax
from jax.experimental import pallas as pl
from jax.experimental.pallas import tpu as pltpu
from jax.experimental.pallas import tpu_sc as plsc

NC = 2
NS = 16
NW = NC * NS
IDX_CHUNK = 128
LANES = 16
WIN = 1024


def _stream_body(eht_hbm, col_hbm, flag_hbm, nxt_hbm, par_hbm, w0_hbm,
                 out_hbm, col_v, flag_v, nxt_v, par_v, w0_v,
                 buf0, buf1, soa_v, sem0, sem1):
  n_ent = col_v.shape[0]
  dim = eht_hbm.shape[0]
  wid = lax.axis_index("s") * NC + lax.axis_index("c")
  pltpu.sync_copy(col_hbm.at[wid], col_v)
  pltpu.sync_copy(flag_hbm.at[wid], flag_v)
  pltpu.sync_copy(nxt_hbm.at[wid], nxt_v)
  pltpu.sync_copy(par_hbm.at[wid], par_v)
  pltpu.sync_copy(w0_hbm.at[wid], w0_v)

  iota = lax.iota(jnp.int32, LANES)
  zeros = jnp.zeros((LANES,), jnp.int32)

  w0 = w0_v[0:LANES][0]
  pltpu.async_copy(eht_hbm.at[:, pl.ds(pl.multiple_of(w0, 128), WIN)],
                   buf0, sem0)

  def drain(buf, sem):
    pltpu.make_async_copy(eht_hbm.at[:, pl.ds(0, WIN)], buf, sem).wait()

  def group(g, carry):
    cvec = col_v[pl.ds(g * LANES, LANES)]
    fvec = flag_v[pl.ds(g * LANES, LANES)]
    nvec = nxt_v[pl.ds(g * LANES, LANES)]
    pvec = par_v[pl.ds(g * LANES, LANES)]
    for j in range(LANES):
      col = cvec[j]
      f = fvec[j]
      nxt = nvec[j]
      p = pvec[j]

      @pl.when((f == 1) & (p == 0))
      def _():
        drain(buf0, sem0)
        pltpu.async_copy(
            eht_hbm.at[:, pl.ds(pl.multiple_of(nxt, 128), WIN)], buf1, sem1)

      @pl.when((f == 1) & (p == 1))
      def _():
        drain(buf1, sem1)
        pltpu.async_copy(
            eht_hbm.at[:, pl.ds(pl.multiple_of(nxt, 128), WIN)], buf0, sem0)

      colv = zeros + col
      k = zeros + (g * LANES + j)

      @pl.when(p == 0)
      def _():
        lo = plsc.load_gather(buf0, [iota, colv])
        hi = plsc.load_gather(buf0, [iota + LANES, colv])
        plsc.store_scatter(soa_v, [iota, k], lo)
        plsc.store_scatter(soa_v, [iota + LANES, k], hi)

      @pl.when(p == 1)
      def _():
        lo = plsc.load_gather(buf1, [iota, colv])
        hi = plsc.load_gather(buf1, [iota + LANES, colv])
        plsc.store_scatter(soa_v, [iota, k], lo)
        plsc.store_scatter(soa_v, [iota + LANES, k], hi)
    return carry

  lax.fori_loop(0, n_ent // LANES, group, jnp.int32(0))

  fin = w0_v[0:LANES][1]

  @pl.when(fin == 0)
  def _():
    drain(buf0, sem0)

  @pl.when(fin == 1)
  def _():
    drain(buf1, sem1)

  pltpu.sync_copy(soa_v, out_hbm.at[:, pl.ds(wid * n_ent, n_ent)])


@jax.jit
def _sc_stream_gather(EhT, col3, flag3, nxt3, par3, w03):
  nw, n_ent = col3.shape
  dim = EhT.shape[0]
  mesh = plsc.VectorSubcoreMesh(core_axis_name="c", subcore_axis_name="s")
  ient = pltpu.VMEM((n_ent,), jnp.int32)
  run = pl.kernel(
      _stream_body,
      mesh=mesh,
      compiler_params=pltpu.CompilerParams(use_tc_tiling_on_sc=True,
                                           needs_layout_passes=False,
                                           disable_bounds_checks=True),
      out_type=[jax.ShapeDtypeStruct((dim, nw * n_ent), jnp.float32)],
      scratch_types=[
          ient, ient, ient, ient,
          pltpu.VMEM((LANES,), jnp.int32),
          pltpu.VMEM((dim, WIN), jnp.float32),
          pltpu.VMEM((dim, WIN), jnp.float32),
          pltpu.VMEM((dim, n_ent), jnp.float32),
          pltpu.SemaphoreType.DMA,
          pltpu.SemaphoreType.DMA,
      ],
  )
  return run(EhT, col3, flag3, nxt3, par3, w03)


def _gather_body(order_hbm, ridx_hbm, uhi_hbm, vhi_hbm,
                 ulo_hbm, vlo_hbm, uv_hbm, rvh_hbm, wu_hbm,
                 bs_hbm, bo_hbm,
                 uv_out, ru_out, rv_out, bsg_out, bog_out,
                 order_v, ridx_v, uhi_v, vhi_v, ulo_v, vlo_v,
                 uv_rows, ru_v, rv_v, bsr_v, bor_v, bsg_v, bog_v, sem):
  n_chunks = ridx_v.shape[0]
  bpw = n_chunks * IDX_CHUNK
  uv_chunks = order_v.shape[0]
  upw = uv_chunks * IDX_CHUNK
  wid = lax.axis_index("s") * NC + lax.axis_index("c")
  base = wid * bpw

  pltpu.sync_copy(order_hbm.at[wid], order_v)
  pltpu.sync_copy(ridx_hbm.at[wid], ridx_v)
  pltpu.sync_copy(uhi_hbm.at[wid], uhi_v)
  pltpu.sync_copy(vhi_hbm.at[wid], vhi_v)
  pltpu.sync_copy(ulo_hbm.at[wid], ulo_v)
  pltpu.sync_copy(vlo_hbm.at[wid], vlo_v)
  pltpu.sync_copy(uv_hbm.at[pl.ds(wid * upw, upw)], uv_rows)

  copies = []
  for c in range(uv_chunks):
    rows = pl.ds(c * IDX_CHUNK, IDX_CHUNK)
    copies.append(pltpu.async_copy(uv_rows.at[rows],
                                   uv_out.at[order_v.at[c]], sem))
  for c in range(n_chunks):
    rows = pl.ds(c * IDX_CHUNK, IDX_CHUNK)
    copies.append(pltpu.async_copy(wu_hbm.at[ridx_v.at[c]], ru_v.at[rows], sem))
    copies.append(pltpu.async_copy(rvh_hbm.at[ridx_v.at[c]], rv_v.at[rows], sem))
    copies.append(pltpu.async_copy(bs_hbm.at[uhi_v.at[c]], bsr_v.at[rows], sem))
    copies.append(pltpu.async_copy(bo_hbm.at[vhi_v.at[c]], bor_v.at[rows], sem))
  for cp in copies:
    cp.wait()

  for g in range(bpw // LANES):
    row_ids = g * LANES + lax.iota(jnp.int32, LANES)
    c = (g * LANES) // IDX_CHUNK
    o = (g * LANES) % IDX_CHUNK
    ucol = ulo_v[c, pl.ds(o, LANES)]
    vcol = vlo_v[c, pl.ds(o, LANES)]
    bsg_v[pl.ds(g * LANES, LANES)] = plsc.load_gather(bsr_v, [row_ids, ucol])
    bog_v[pl.ds(g * LANES, LANES)] = plsc.load_gather(bor_v, [row_ids, vcol])

  out_rows = pl.ds(base, bpw)
  pltpu.sync_copy(ru_v, ru_out.at[out_rows])
  pltpu.sync_copy(rv_v, rv_out.at[out_rows])
  pltpu.sync_copy(bsg_v, bsg_out.at[out_rows])
  pltpu.sync_copy(bog_v, bog_out.at[out_rows])


@jax.jit
def _sc_gather(order3, r_idx3, u_hi3, v_hi3, u_lo3, v_lo3,
               uv2, rvh, Wu, bs16, bo16):
  nw, n_chunks, _ = r_idx3.shape
  uv_chunks = order3.shape[1]
  bpw = n_chunks * IDX_CHUNK
  upw = uv_chunks * IDX_CHUNK
  b = nw * bpw
  dim = uv2.shape[1]
  f32 = jnp.float32
  i32 = jnp.int32
  mesh = plsc.VectorSubcoreMesh(core_axis_name="c", subcore_axis_name="s")
  idx_t = pltpu.VMEM((n_chunks, IDX_CHUNK), i32)
  run = pl.kernel(
      _gather_body,
      mesh=mesh,
      compiler_params=pltpu.CompilerParams(use_tc_tiling_on_sc=False,
                                           needs_layout_passes=False),
      out_type=[
          jax.ShapeDtypeStruct((2 * b, dim), f32),
          jax.ShapeDtypeStruct((b, dim), f32),
          jax.ShapeDtypeStruct((b, dim), f32),
          jax.ShapeDtypeStruct((b,), f32),
          jax.ShapeDtypeStruct((b,), f32),
      ],
      scratch_types=[
          pltpu.VMEM((uv_chunks, IDX_CHUNK), i32),
          idx_t, idx_t, idx_t, idx_t, idx_t,
          pltpu.VMEM((upw, dim), f32),
          pltpu.VMEM((bpw, dim), f32),
          pltpu.VMEM((bpw, dim), f32),
          pltpu.VMEM((bpw, LANES), f32),
          pltpu.VMEM((bpw, LANES), f32),
          pltpu.VMEM((bpw,), f32),
          pltpu.VMEM((bpw,), f32),
          pltpu.SemaphoreType.DMA,
      ],
  )
  return run(order3, r_idx3, u_hi3, v_hi3, u_lo3, v_lo3,
             uv2, rvh, Wu, bs16, bo16)


def _artanh(x):
  return 0.5 * jnp.log((1 + x) / (1 - x))


def _sqnorm(x):
  return jnp.sum(x * x, axis=0, keepdims=True)


def _norm(x):
  return jnp.sqrt(_sqnorm(x))


def _proj(t, eps=1e-5):
  nrm = _norm(t)
  msk = (nrm >= 1).astype(t.dtype)
  return t / (nrm - eps) * msk + t * (1 - msk)


def _p_sum(x, y):
  sqxnorm = jnp.clip(_sqnorm(x), 0.0, 1 - 1e-5)
  sqynorm = jnp.clip(_sqnorm(y), 0.0, 1 - 1e-5)
  dotxy = jnp.sum(x * y, axis=0, keepdims=True)
  numerator = (1 + 2 * dotxy + sqynorm) * x + (1 - sqxnorm) * y
  denominator = 1 + 2 * dotxy + sqxnorm * sqynorm
  return numerator / denominator


def _math_body(u_ref, v_ref, ru_ref, rv_ref, bs_ref, bo_ref, out_ref):
  u = _proj(u_ref[...].T)
  v = _proj(v_ref[...].T)
  rvh_g = _proj(rv_ref[...].T)
  Ru = ru_ref[...].T

  normu = jnp.clip(_norm(u), 1e-10, 1 - 1e-5)
  u_e = _artanh(normu) * u / normu
  u_W = u_e * Ru
  normw = jnp.clip(_norm(u_W), 1e-10, None)
  u_m = jnp.tanh(normw) * u_W / normw
  v_m = _p_sum(v, rvh_g)
  u_m = _proj(u_m)
  v_m = _proj(v_m)
  d = _p_sum(-u_m, v_m)
  nrm = jnp.clip(jnp.sqrt(jnp.sum(d * d, axis=0)), 1e-10, 1 - 1e-5)
  sqdist = (2.0 * _artanh(nrm)) ** 2
  out_ref[...] = -sqdist + bs_ref[...][:, 0] + bo_ref[...][:, 0]


@jax.jit
def _tc_math(u, v, ru, rv, bsg, bog):
  b, dim = u.shape
  blk = 2048
  grid = (b // blk,)
  row_spec = pl.BlockSpec((blk, dim), lambda i: (i, 0))
  one_spec = pl.BlockSpec((blk, 1), lambda i: (i, 0))
  return pl.pallas_call(
      _math_body,
      grid=grid,
      in_specs=[row_spec, row_spec, row_spec, row_spec, one_spec, one_spec],
      out_specs=pl.BlockSpec((blk,), lambda i: (i,)),
      out_shape=jax.ShapeDtypeStruct((b,), jnp.float32),
  )(u, v, ru, rv, bsg, bog)


def kernel(u_idx, r_idx, v_idx, Eh, rvh, Wu, bs, bo):
  b = u_idx.shape[0]
  n_chunks = b // (NW * IDX_CHUNK)
  shape3 = (NW, n_chunks, IDX_CHUNK)
  u_idx = u_idx.astype(jnp.int32)
  r_idx = r_idx.astype(jnp.int32)
  v_idx = v_idx.astype(jnp.int32)

  ent = jnp.concatenate([u_idx, v_idx])
  order = jnp.argsort(ent).astype(jnp.int32)
  ents_sorted = jnp.take(ent, order)
  order3 = order.reshape(NW, 2 * n_chunks, IDX_CHUNK)

  n_ent = (2 * b) // NW
  pad_minor = ((Eh.shape[0] + 127) // 128) * 128
  base2 = jnp.minimum((ents_sorted >> 10) << 10,
                      pad_minor - WIN).reshape(NW, n_ent)
  col2 = ents_sorted.reshape(NW, n_ent) - base2
  flag2 = jnp.concatenate(
      [jnp.ones((NW, 1), jnp.int32),
       (base2[:, 1:] != base2[:, :-1]).astype(jnp.int32)], axis=1)
  ord2 = jnp.cumsum(flag2, axis=1) - 1
  par2 = ord2 & 1
  nwin = ord2[:, -1] + 1
  rowix = jnp.broadcast_to(jnp.arange(NW)[:, None], (NW, n_ent))
  win_seq = jnp.zeros((NW, n_ent), jnp.int32).at[
      rowix, jnp.where(flag2 == 1, ord2, n_ent)].set(base2, mode='drop')
  nxt2 = jnp.take_along_axis(
      win_seq, jnp.minimum(ord2 + 1, (nwin - 1)[:, None]), axis=1)
  w03 = jnp.zeros((NW, LANES), jnp.int32)
  w03 = w03.at[:, 0].set(base2[:, 0]).at[:, 1].set(nwin & 1)

  EhT = jnp.swapaxes(Eh, 0, 1)
  uv_sorted = _sc_stream_gather(EhT, col2, flag2, nxt2, par2, w03)[0]
  uv2 = jnp.swapaxes(uv_sorted, 0, 1)

  r_idx3 = r_idx.reshape(shape3)
  u_hi3 = (u_idx >> 4).reshape(shape3)
  v_hi3 = (v_idx >> 4).reshape(shape3)
  u_lo3 = (u_idx & (LANES - 1)).reshape(shape3)
  v_lo3 = (v_idx & (LANES - 1)).reshape(shape3)
  bs16 = bs.reshape(-1, LANES)
  bo16 = bo.reshape(-1, LANES)
  uv_unperm, ru, rv, bsg, bog = _sc_gather(order3, r_idx3,
                                           u_hi3, v_hi3, u_lo3, v_lo3,
                                           uv2, rvh, Wu, bs16, bo16)
  u = uv_unperm[:b]
  v = uv_unperm[b:]
  return _tc_math(u, v, ru, rv, bsg[:, None], bog[:, None])


# speedup vs baseline: 1.6110x; 1.6110x over previous
import functools

import jax
import jax.numpy as jnp
from jax import lax
from jax.experimental import pallas as pl
from jax.experimental.pallas import tpu as pltpu
from jax.experimental.pallas import tpu_sc as plsc

NC = 2
NS = 16
NW = NC * NS
IDX_CHUNK = 128
LANES = 16
WIN = 2048


def _stream_body(eht_hbm, ents_hbm, out_hbm, ent_v, win_v, soa_v):
  n_ent = ent_v.shape[0]
  dim, n_table = eht_hbm.shape
  pad_minor = ((n_table + 127) // 128) * 128
  cb_max = (pad_minor - WIN) // 128
  wid = lax.axis_index("s") * NC + lax.axis_index("c")
  pltpu.sync_copy(ents_hbm.at[wid], ent_v)

  iota = lax.iota(jnp.int32, LANES)
  zeros = jnp.zeros((LANES,), jnp.int32)

  def group(g, cb):
    evec = ent_v[pl.ds(g * LANES, LANES)]
    for j in range(LANES):
      e = evec[j]
      trig = (e - cb * 128) >= WIN
      newcb = jnp.minimum(lax.shift_right_logical(e, 7), cb_max)
      cb = jnp.where(trig, newcb, cb)

      @pl.when(trig)
      def _():
        off = pl.multiple_of(cb * 128, 128)
        pltpu.sync_copy(eht_hbm.at[:, pl.ds(off, WIN)], win_v)

      col = zeros + (e - cb * 128)
      k = zeros + (g * LANES + j)
      lo = plsc.load_gather(win_v, [iota, col])
      hi = plsc.load_gather(win_v, [iota + LANES, col])
      plsc.store_scatter(soa_v, [iota, k], lo)
      plsc.store_scatter(soa_v, [iota + LANES, k], hi)
    return cb

  lax.fori_loop(0, n_ent // LANES, group, jnp.int32(-(2 ** 20)))
  pltpu.sync_copy(soa_v, out_hbm.at[:, pl.ds(wid * n_ent, n_ent)])


@jax.jit
def _sc_stream_gather(EhT, ents2):
  nw, n_ent = ents2.shape
  dim = EhT.shape[0]
  mesh = plsc.VectorSubcoreMesh(core_axis_name="c", subcore_axis_name="s")
  run = pl.kernel(
      _stream_body,
      mesh=mesh,
      compiler_params=pltpu.CompilerParams(use_tc_tiling_on_sc=True,
                                           needs_layout_passes=False,
                                           disable_bounds_checks=True),
      out_type=[jax.ShapeDtypeStruct((dim, nw * n_ent), jnp.float32)],
      scratch_types=[
          pltpu.VMEM((n_ent,), jnp.int32),
          pltpu.VMEM((dim, WIN), jnp.float32),
          pltpu.VMEM((dim, n_ent), jnp.float32),
      ],
  )
  return run(EhT, ents2)


def _gather_body(order_hbm, ridx_hbm, uhi_hbm, vhi_hbm,
                 ulo_hbm, vlo_hbm, uv_hbm, rvh_hbm, wu_hbm,
                 bs_hbm, bo_hbm,
                 uv_out, ru_out, rv_out, bsg_out, bog_out,
                 order_v, ridx_v, uhi_v, vhi_v, ulo_v, vlo_v,
                 uv_rows, ru_v, rv_v, bsr_v, bor_v, bsg_v, bog_v, sem):
  n_chunks = ridx_v.shape[0]
  bpw = n_chunks * IDX_CHUNK
  uv_chunks = order_v.shape[0]
  upw = uv_chunks * IDX_CHUNK
  wid = lax.axis_index("s") * NC + lax.axis_index("c")
  base = wid * bpw

  pltpu.sync_copy(order_hbm.at[wid], order_v)
  pltpu.sync_copy(ridx_hbm.at[wid], ridx_v)
  pltpu.sync_copy(uhi_hbm.at[wid], uhi_v)
  pltpu.sync_copy(vhi_hbm.at[wid], vhi_v)
  pltpu.sync_copy(ulo_hbm.at[wid], ulo_v)
  pltpu.sync_copy(vlo_hbm.at[wid], vlo_v)
  pltpu.sync_copy(uv_hbm.at[pl.ds(wid * upw, upw)], uv_rows)

  copies = []
  for c in range(uv_chunks):
    rows = pl.ds(c * IDX_CHUNK, IDX_CHUNK)
    copies.append(pltpu.async_copy(uv_rows.at[rows],
                                   uv_out.at[order_v.at[c]], sem))
  for c in range(n_chunks):
    rows = pl.ds(c * IDX_CHUNK, IDX_CHUNK)
    copies.append(pltpu.async_copy(wu_hbm.at[ridx_v.at[c]], ru_v.at[rows], sem))
    copies.append(pltpu.async_copy(rvh_hbm.at[ridx_v.at[c]], rv_v.at[rows], sem))
    copies.append(pltpu.async_copy(bs_hbm.at[uhi_v.at[c]], bsr_v.at[rows], sem))
    copies.append(pltpu.async_copy(bo_hbm.at[vhi_v.at[c]], bor_v.at[rows], sem))
  for cp in copies:
    cp.wait()

  for g in range(bpw // LANES):
    row_ids = g * LANES + lax.iota(jnp.int32, LANES)
    c = (g * LANES) // IDX_CHUNK
    o = (g * LANES) % IDX_CHUNK
    ucol = ulo_v[c, pl.ds(o, LANES)]
    vcol = vlo_v[c, pl.ds(o, LANES)]
    bsg_v[pl.ds(g * LANES, LANES)] = plsc.load_gather(bsr_v, [row_ids, ucol])
    bog_v[pl.ds(g * LANES, LANES)] = plsc.load_gather(bor_v, [row_ids, vcol])

  out_rows = pl.ds(base, bpw)
  pltpu.sync_copy(ru_v, ru_out.at[out_rows])
  pltpu.sync_copy(rv_v, rv_out.at[out_rows])
  pltpu.sync_copy(bsg_v, bsg_out.at[out_rows])
  pltpu.sync_copy(bog_v, bog_out.at[out_rows])


@jax.jit
def _sc_gather(order3, r_idx3, u_hi3, v_hi3, u_lo3, v_lo3,
               uv2, rvh, Wu, bs16, bo16):
  nw, n_chunks, _ = r_idx3.shape
  uv_chunks = order3.shape[1]
  bpw = n_chunks * IDX_CHUNK
  upw = uv_chunks * IDX_CHUNK
  b = nw * bpw
  dim = uv2.shape[1]
  f32 = jnp.float32
  i32 = jnp.int32
  mesh = plsc.VectorSubcoreMesh(core_axis_name="c", subcore_axis_name="s")
  idx_t = pltpu.VMEM((n_chunks, IDX_CHUNK), i32)
  run = pl.kernel(
      _gather_body,
      mesh=mesh,
      compiler_params=pltpu.CompilerParams(use_tc_tiling_on_sc=False,
                                           needs_layout_passes=False),
      out_type=[
          jax.ShapeDtypeStruct((2 * b, dim), f32),
          jax.ShapeDtypeStruct((b, dim), f32),
          jax.ShapeDtypeStruct((b, dim), f32),
          jax.ShapeDtypeStruct((b,), f32),
          jax.ShapeDtypeStruct((b,), f32),
      ],
      scratch_types=[
          pltpu.VMEM((uv_chunks, IDX_CHUNK), i32),
          idx_t, idx_t, idx_t, idx_t, idx_t,
          pltpu.VMEM((upw, dim), f32),
          pltpu.VMEM((bpw, dim), f32),
          pltpu.VMEM((bpw, dim), f32),
          pltpu.VMEM((bpw, LANES), f32),
          pltpu.VMEM((bpw, LANES), f32),
          pltpu.VMEM((bpw,), f32),
          pltpu.VMEM((bpw,), f32),
          pltpu.SemaphoreType.DMA,
      ],
  )
  return run(order3, r_idx3, u_hi3, v_hi3, u_lo3, v_lo3,
             uv2, rvh, Wu, bs16, bo16)


def _artanh(x):
  return 0.5 * jnp.log((1 + x) / (1 - x))


def _sqnorm(x):
  return jnp.sum(x * x, axis=0, keepdims=True)


def _norm(x):
  return jnp.sqrt(_sqnorm(x))


def _proj(t, eps=1e-5):
  nrm = _norm(t)
  msk = (nrm >= 1).astype(t.dtype)
  return t / (nrm - eps) * msk + t * (1 - msk)


def _p_sum(x, y):
  sqxnorm = jnp.clip(_sqnorm(x), 0.0, 1 - 1e-5)
  sqynorm = jnp.clip(_sqnorm(y), 0.0, 1 - 1e-5)
  dotxy = jnp.sum(x * y, axis=0, keepdims=True)
  numerator = (1 + 2 * dotxy + sqynorm) * x + (1 - sqxnorm) * y
  denominator = 1 + 2 * dotxy + sqxnorm * sqynorm
  return numerator / denominator


def _math_body(u_ref, v_ref, ru_ref, rv_ref, bs_ref, bo_ref, out_ref):
  u = _proj(u_ref[...].T)
  v = _proj(v_ref[...].T)
  rvh_g = _proj(rv_ref[...].T)
  Ru = ru_ref[...].T

  normu = jnp.clip(_norm(u), 1e-10, 1 - 1e-5)
  u_e = _artanh(normu) * u / normu
  u_W = u_e * Ru
  normw = jnp.clip(_norm(u_W), 1e-10, None)
  u_m = jnp.tanh(normw) * u_W / normw
  v_m = _p_sum(v, rvh_g)
  u_m = _proj(u_m)
  v_m = _proj(v_m)
  d = _p_sum(-u_m, v_m)
  nrm = jnp.clip(jnp.sqrt(jnp.sum(d * d, axis=0)), 1e-10, 1 - 1e-5)
  sqdist = (2.0 * _artanh(nrm)) ** 2
  out_ref[...] = -sqdist + bs_ref[...][:, 0] + bo_ref[...][:, 0]


@jax.jit
def _tc_math(u, v, ru, rv, bsg, bog):
  b, dim = u.shape
  blk = 8192
  grid = (b // blk,)
  row_spec = pl.BlockSpec((blk, dim), lambda i: (i, 0))
  one_spec = pl.BlockSpec((blk, 1), lambda i: (i, 0))
  return pl.pallas_call(
      _math_body,
      grid=grid,
      in_specs=[row_spec, row_spec, row_spec, row_spec, one_spec, one_spec],
      out_specs=pl.BlockSpec((blk,), lambda i: (i,)),
      out_shape=jax.ShapeDtypeStruct((b,), jnp.float32),
  )(u, v, ru, rv, bsg, bog)


def kernel(u_idx, r_idx, v_idx, Eh, rvh, Wu, bs, bo):
  b = u_idx.shape[0]
  n_chunks = b // (NW * IDX_CHUNK)
  shape3 = (NW, n_chunks, IDX_CHUNK)
  u_idx = u_idx.astype(jnp.int32)
  r_idx = r_idx.astype(jnp.int32)
  v_idx = v_idx.astype(jnp.int32)

  ent = jnp.concatenate([u_idx, v_idx])
  order = jnp.argsort(ent).astype(jnp.int32)
  ents_sorted = jnp.take(ent, order)
  order3 = order.reshape(NW, 2 * n_chunks, IDX_CHUNK)
  ents2 = ents_sorted.reshape(NW, (2 * b) // NW)

  EhT = jnp.swapaxes(Eh, 0, 1)
  uv_sorted = _sc_stream_gather(EhT, ents2)[0]
  uv2 = jnp.swapaxes(uv_sorted, 0, 1)

  r_idx3 = r_idx.reshape(shape3)
  u_hi3 = (u_idx >> 4).reshape(shape3)
  v_hi3 = (v_idx >> 4).reshape(shape3)
  u_lo3 = (u_idx & (LANES - 1)).reshape(shape3)
  v_lo3 = (v_idx & (LANES - 1)).reshape(shape3)
  bs16 = bs.reshape(-1, LANES)
  bo16 = bo.reshape(-1, LANES)
  uv_unperm, ru, rv, bsg, bog = _sc_gather(order3, r_idx3,
                                           u_hi3, v_hi3, u_lo3, v_lo3,
                                           uv2, rvh, Wu, bs16, bo16)
  u = uv_unperm[:b]
  v = uv_unperm[b:]
  return _tc_math(u, v, ru, rv, bsg[:, None], bog[:, None])


# speedup vs baseline: 1.6383x; 1.0170x over previous
import functools

import jax
import jax.numpy as jnp
from jax import lax
from jax.experimental import pallas as pl
from jax.experimental.pallas import tpu as pltpu
from jax.experimental.pallas import tpu_sc as plsc

NC = 2
NS = 16
NW = NC * NS
IDX_CHUNK = 128
LANES = 16
WIN = 2560


def _stream_body(eht_hbm, ents_hbm, out_hbm, ent_v, win_v, soa_v):
  n_ent = ent_v.shape[0]
  dim, n_table = eht_hbm.shape
  pad_minor = ((n_table + 127) // 128) * 128
  cb_max = (pad_minor - WIN) // 128
  wid = lax.axis_index("s") * NC + lax.axis_index("c")
  pltpu.sync_copy(ents_hbm.at[wid], ent_v)

  iota = lax.iota(jnp.int32, LANES)
  zeros = jnp.zeros((LANES,), jnp.int32)

  def group(g, cb):
    evec = ent_v[pl.ds(g * LANES, LANES)]
    for j in range(LANES):
      e = evec[j]
      trig = (e - cb * 128) >= WIN
      newcb = jnp.minimum(lax.shift_right_logical(e, 7), cb_max)
      cb = jnp.where(trig, newcb, cb)

      @pl.when(trig)
      def _():
        off = pl.multiple_of(cb * 128, 128)
        pltpu.sync_copy(eht_hbm.at[:, pl.ds(off, WIN)], win_v)

      col = zeros + (e - cb * 128)
      k = zeros + (g * LANES + j)
      lo = plsc.load_gather(win_v, [iota, col])
      hi = plsc.load_gather(win_v, [iota + LANES, col])
      plsc.store_scatter(soa_v, [iota, k], lo)
      plsc.store_scatter(soa_v, [iota + LANES, k], hi)
    return cb

  lax.fori_loop(0, n_ent // LANES, group, jnp.int32(-(2 ** 20)))
  pltpu.sync_copy(soa_v, out_hbm.at[:, pl.ds(wid * n_ent, n_ent)])


@jax.jit
def _sc_stream_gather(EhT, ents2):
  nw, n_ent = ents2.shape
  dim = EhT.shape[0]
  mesh = plsc.VectorSubcoreMesh(core_axis_name="c", subcore_axis_name="s")
  run = pl.kernel(
      _stream_body,
      mesh=mesh,
      compiler_params=pltpu.CompilerParams(use_tc_tiling_on_sc=True,
                                           needs_layout_passes=False,
                                           disable_bounds_checks=True),
      out_type=[jax.ShapeDtypeStruct((dim, nw * n_ent), jnp.float32)],
      scratch_types=[
          pltpu.VMEM((n_ent,), jnp.int32),
          pltpu.VMEM((dim, WIN), jnp.float32),
          pltpu.VMEM((dim, n_ent), jnp.float32),
      ],
  )
  return run(EhT, ents2)


def _gather_body(order_hbm, ridx_hbm, uhi_hbm, vhi_hbm,
                 ulo_hbm, vlo_hbm, uv_hbm, rvh_hbm, wu_hbm,
                 bs_hbm, bo_hbm,
                 uv_out, ru_out, rv_out, bsg_out, bog_out,
                 order_v, ridx_v, uhi_v, vhi_v, ulo_v, vlo_v,
                 uv_rows, ru_v, rv_v, bsr_v, bor_v, bsg_v, bog_v, sem):
  n_chunks = ridx_v.shape[0]
  bpw = n_chunks * IDX_CHUNK
  uv_chunks = order_v.shape[0]
  upw = uv_chunks * IDX_CHUNK
  wid = lax.axis_index("s") * NC + lax.axis_index("c")
  base = wid * bpw

  pltpu.sync_copy(order_hbm.at[wid], order_v)
  pltpu.sync_copy(ridx_hbm.at[wid], ridx_v)
  pltpu.sync_copy(uhi_hbm.at[wid], uhi_v)
  pltpu.sync_copy(vhi_hbm.at[wid], vhi_v)
  pltpu.sync_copy(ulo_hbm.at[wid], ulo_v)
  pltpu.sync_copy(vlo_hbm.at[wid], vlo_v)
  pltpu.sync_copy(uv_hbm.at[pl.ds(wid * upw, upw)], uv_rows)

  copies = []
  for c in range(uv_chunks):
    rows = pl.ds(c * IDX_CHUNK, IDX_CHUNK)
    copies.append(pltpu.async_copy(uv_rows.at[rows],
                                   uv_out.at[order_v.at[c]], sem))
  for c in range(n_chunks):
    rows = pl.ds(c * IDX_CHUNK, IDX_CHUNK)
    copies.append(pltpu.async_copy(wu_hbm.at[ridx_v.at[c]], ru_v.at[rows], sem))
    copies.append(pltpu.async_copy(rvh_hbm.at[ridx_v.at[c]], rv_v.at[rows], sem))
    copies.append(pltpu.async_copy(bs_hbm.at[uhi_v.at[c]], bsr_v.at[rows], sem))
    copies.append(pltpu.async_copy(bo_hbm.at[vhi_v.at[c]], bor_v.at[rows], sem))
  for cp in copies:
    cp.wait()

  for g in range(bpw // LANES):
    row_ids = g * LANES + lax.iota(jnp.int32, LANES)
    c = (g * LANES) // IDX_CHUNK
    o = (g * LANES) % IDX_CHUNK
    ucol = ulo_v[c, pl.ds(o, LANES)]
    vcol = vlo_v[c, pl.ds(o, LANES)]
    bsg_v[pl.ds(g * LANES, LANES)] = plsc.load_gather(bsr_v, [row_ids, ucol])
    bog_v[pl.ds(g * LANES, LANES)] = plsc.load_gather(bor_v, [row_ids, vcol])

  out_rows = pl.ds(base, bpw)
  pltpu.sync_copy(ru_v, ru_out.at[out_rows])
  pltpu.sync_copy(rv_v, rv_out.at[out_rows])
  pltpu.sync_copy(bsg_v, bsg_out.at[out_rows])
  pltpu.sync_copy(bog_v, bog_out.at[out_rows])


@jax.jit
def _sc_gather(order3, r_idx3, u_hi3, v_hi3, u_lo3, v_lo3,
               uv2, rvh, Wu, bs16, bo16):
  nw, n_chunks, _ = r_idx3.shape
  uv_chunks = order3.shape[1]
  bpw = n_chunks * IDX_CHUNK
  upw = uv_chunks * IDX_CHUNK
  b = nw * bpw
  dim = uv2.shape[1]
  f32 = jnp.float32
  i32 = jnp.int32
  mesh = plsc.VectorSubcoreMesh(core_axis_name="c", subcore_axis_name="s")
  idx_t = pltpu.VMEM((n_chunks, IDX_CHUNK), i32)
  run = pl.kernel(
      _gather_body,
      mesh=mesh,
      compiler_params=pltpu.CompilerParams(use_tc_tiling_on_sc=False,
                                           needs_layout_passes=False),
      out_type=[
          jax.ShapeDtypeStruct((2 * b, dim), f32),
          jax.ShapeDtypeStruct((b, dim), f32),
          jax.ShapeDtypeStruct((b, dim), f32),
          jax.ShapeDtypeStruct((b,), f32),
          jax.ShapeDtypeStruct((b,), f32),
      ],
      scratch_types=[
          pltpu.VMEM((uv_chunks, IDX_CHUNK), i32),
          idx_t, idx_t, idx_t, idx_t, idx_t,
          pltpu.VMEM((upw, dim), f32),
          pltpu.VMEM((bpw, dim), f32),
          pltpu.VMEM((bpw, dim), f32),
          pltpu.VMEM((bpw, LANES), f32),
          pltpu.VMEM((bpw, LANES), f32),
          pltpu.VMEM((bpw,), f32),
          pltpu.VMEM((bpw,), f32),
          pltpu.SemaphoreType.DMA,
      ],
  )
  return run(order3, r_idx3, u_hi3, v_hi3, u_lo3, v_lo3,
             uv2, rvh, Wu, bs16, bo16)


def _artanh(x):
  return 0.5 * jnp.log((1 + x) / (1 - x))


def _sqnorm(x):
  return jnp.sum(x * x, axis=0, keepdims=True)


def _norm(x):
  return jnp.sqrt(_sqnorm(x))


def _proj(t, eps=1e-5):
  nrm = _norm(t)
  msk = (nrm >= 1).astype(t.dtype)
  return t / (nrm - eps) * msk + t * (1 - msk)


def _p_sum(x, y):
  sqxnorm = jnp.clip(_sqnorm(x), 0.0, 1 - 1e-5)
  sqynorm = jnp.clip(_sqnorm(y), 0.0, 1 - 1e-5)
  dotxy = jnp.sum(x * y, axis=0, keepdims=True)
  numerator = (1 + 2 * dotxy + sqynorm) * x + (1 - sqxnorm) * y
  denominator = 1 + 2 * dotxy + sqxnorm * sqynorm
  return numerator / denominator


def _math_body(u_ref, v_ref, ru_ref, rv_ref, bs_ref, bo_ref, out_ref):
  u = _proj(u_ref[...].T)
  v = _proj(v_ref[...].T)
  rvh_g = _proj(rv_ref[...].T)
  Ru = ru_ref[...].T

  normu = jnp.clip(_norm(u), 1e-10, 1 - 1e-5)
  u_e = _artanh(normu) * u / normu
  u_W = u_e * Ru
  normw = jnp.clip(_norm(u_W), 1e-10, None)
  u_m = jnp.tanh(normw) * u_W / normw
  v_m = _p_sum(v, rvh_g)
  u_m = _proj(u_m)
  v_m = _proj(v_m)
  d = _p_sum(-u_m, v_m)
  nrm = jnp.clip(jnp.sqrt(jnp.sum(d * d, axis=0)), 1e-10, 1 - 1e-5)
  sqdist = (2.0 * _artanh(nrm)) ** 2
  out_ref[...] = -sqdist + bs_ref[...][:, 0] + bo_ref[...][:, 0]


@jax.jit
def _tc_math(u, v, ru, rv, bsg, bog):
  b, dim = u.shape
  blk = 2048
  grid = (b // blk,)
  row_spec = pl.BlockSpec((blk, dim), lambda i: (i, 0))
  one_spec = pl.BlockSpec((blk, 1), lambda i: (i, 0))
  return pl.pallas_call(
      _math_body,
      grid=grid,
      in_specs=[row_spec, row_spec, row_spec, row_spec, one_spec, one_spec],
      out_specs=pl.BlockSpec((blk,), lambda i: (i,)),
      out_shape=jax.ShapeDtypeStruct((b,), jnp.float32),
  )(u, v, ru, rv, bsg, bog)


def kernel(u_idx, r_idx, v_idx, Eh, rvh, Wu, bs, bo):
  b = u_idx.shape[0]
  n_chunks = b // (NW * IDX_CHUNK)
  shape3 = (NW, n_chunks, IDX_CHUNK)
  u_idx = u_idx.astype(jnp.int32)
  r_idx = r_idx.astype(jnp.int32)
  v_idx = v_idx.astype(jnp.int32)

  ent = jnp.concatenate([u_idx, v_idx])
  order = jnp.argsort(ent).astype(jnp.int32)
  ents_sorted = jnp.take(ent, order)
  order3 = order.reshape(NW, 2 * n_chunks, IDX_CHUNK)
  ents2 = ents_sorted.reshape(NW, (2 * b) // NW)

  EhT = jnp.swapaxes(Eh, 0, 1)
  uv_sorted = _sc_stream_gather(EhT, ents2)[0]
  uv2 = jnp.swapaxes(uv_sorted, 0, 1)

  r_idx3 = r_idx.reshape(shape3)
  u_hi3 = (u_idx >> 4).reshape(shape3)
  v_hi3 = (v_idx >> 4).reshape(shape3)
  u_lo3 = (u_idx & (LANES - 1)).reshape(shape3)
  v_lo3 = (v_idx & (LANES - 1)).reshape(shape3)
  bs16 = bs.reshape(-1, LANES)
  bo16 = bo.reshape(-1, LANES)
  uv_unperm, ru, rv, bsg, bog = _sc_gather(order3, r_idx3,
                                           u_hi3, v_hi3, u_lo3, v_lo3,
                                           uv2, rvh, Wu, bs16, bo16)
  u = uv_unperm[:b]
  v = uv_unperm[b:]
  return _tc_math(u, v, ru, rv, bsg[:, None], bog[:, None])


# speedup vs baseline: 1.7265x; 1.0538x over previous
import functools

import jax
import jax.numpy as jnp
from jax import lax
from jax.experimental import pallas as pl
from jax.experimental.pallas import tpu as pltpu
from jax.experimental.pallas import tpu_sc as plsc

NC = 2
NS = 16
NW = NC * NS
IDX_CHUNK = 128
LANES = 16
WIN = 2560


def _stream_body(eht_hbm, ents_hbm, out_hbm, ent_v, win_v, soa_v):
  n_ent = ent_v.shape[0]
  dim, n_table = eht_hbm.shape
  pad_minor = ((n_table + 127) // 128) * 128
  cb_max = (pad_minor - WIN) // 128
  wid = lax.axis_index("s") * NC + lax.axis_index("c")
  pltpu.sync_copy(ents_hbm.at[wid], ent_v)

  iota = lax.iota(jnp.int32, LANES)
  zeros = jnp.zeros((LANES,), jnp.int32)

  def group(g, cb):
    evec = ent_v[pl.ds(g * LANES, LANES)]
    for j in range(LANES):
      e = evec[j]
      trig = (e - cb * 128) >= WIN
      newcb = jnp.minimum(lax.shift_right_logical(e, 7), cb_max)
      cb = jnp.where(trig, newcb, cb)

      @pl.when(trig)
      def _():
        off = pl.multiple_of(cb * 128, 128)
        pltpu.sync_copy(eht_hbm.at[:, pl.ds(off, WIN)], win_v)

      col = zeros + (e - cb * 128)
      k = zeros + (g * LANES + j)
      lo = plsc.load_gather(win_v, [iota, col])
      hi = plsc.load_gather(win_v, [iota + LANES, col])
      plsc.store_scatter(soa_v, [iota, k], lo)
      plsc.store_scatter(soa_v, [iota + LANES, k], hi)
    return cb

  lax.fori_loop(0, n_ent // LANES, group, jnp.int32(-(2 ** 20)))
  pltpu.sync_copy(soa_v, out_hbm.at[:, pl.ds(wid * n_ent, n_ent)])


@jax.jit
def _sc_stream_gather(EhT, ents2):
  nw, n_ent = ents2.shape
  dim = EhT.shape[0]
  mesh = plsc.VectorSubcoreMesh(core_axis_name="c", subcore_axis_name="s")
  run = pl.kernel(
      _stream_body,
      mesh=mesh,
      compiler_params=pltpu.CompilerParams(use_tc_tiling_on_sc=True,
                                           needs_layout_passes=False,
                                           disable_bounds_checks=True),
      out_type=[jax.ShapeDtypeStruct((dim, nw * n_ent), jnp.float32)],
      scratch_types=[
          pltpu.VMEM((n_ent,), jnp.int32),
          pltpu.VMEM((dim, WIN), jnp.float32),
          pltpu.VMEM((dim, n_ent), jnp.float32),
      ],
  )
  return run(EhT, ents2)


def _gather_body(order_hbm, ridx_hbm, uhi_hbm, vhi_hbm,
                 ulo_hbm, vlo_hbm, uv_hbm, rvh_hbm, wu_hbm,
                 bs_hbm, bo_hbm,
                 uv_out, ru_out, rv_out, bsg_out, bog_out,
                 order_v, ridx_v, uhi_v, vhi_v, ulo_v, vlo_v,
                 uv_rows, ru_v, rv_v, bsr_v, bor_v, bsg_v, bog_v, sem):
  n_chunks = ridx_v.shape[0]
  bpw = n_chunks * IDX_CHUNK
  uv_chunks = order_v.shape[0]
  upw = uv_chunks * IDX_CHUNK
  wid = lax.axis_index("s") * NC + lax.axis_index("c")
  base = wid * bpw

  pltpu.sync_copy(order_hbm.at[wid], order_v)
  pltpu.sync_copy(ridx_hbm.at[wid], ridx_v)
  pltpu.sync_copy(uhi_hbm.at[wid], uhi_v)
  pltpu.sync_copy(vhi_hbm.at[wid], vhi_v)
  pltpu.sync_copy(ulo_hbm.at[wid], ulo_v)
  pltpu.sync_copy(vlo_hbm.at[wid], vlo_v)
  pltpu.sync_copy(uv_hbm.at[pl.ds(wid * upw, upw)], uv_rows)

  copies = []
  for c in range(uv_chunks):
    rows = pl.ds(c * IDX_CHUNK, IDX_CHUNK)
    copies.append(pltpu.async_copy(uv_rows.at[rows],
                                   uv_out.at[order_v.at[c]], sem))
  for c in range(n_chunks):
    rows = pl.ds(c * IDX_CHUNK, IDX_CHUNK)
    copies.append(pltpu.async_copy(wu_hbm.at[ridx_v.at[c]], ru_v.at[rows], sem))
    copies.append(pltpu.async_copy(rvh_hbm.at[ridx_v.at[c]], rv_v.at[rows], sem))
    copies.append(pltpu.async_copy(bs_hbm.at[uhi_v.at[c]], bsr_v.at[rows], sem))
    copies.append(pltpu.async_copy(bo_hbm.at[vhi_v.at[c]], bor_v.at[rows], sem))
  for cp in copies:
    cp.wait()

  for g in range(bpw // LANES):
    row_ids = g * LANES + lax.iota(jnp.int32, LANES)
    c = (g * LANES) // IDX_CHUNK
    o = (g * LANES) % IDX_CHUNK
    ucol = ulo_v[c, pl.ds(o, LANES)]
    vcol = vlo_v[c, pl.ds(o, LANES)]
    bsg_v[pl.ds(g * LANES, LANES)] = plsc.load_gather(bsr_v, [row_ids, ucol])
    bog_v[pl.ds(g * LANES, LANES)] = plsc.load_gather(bor_v, [row_ids, vcol])

  out_rows = pl.ds(base, bpw)
  pltpu.sync_copy(ru_v, ru_out.at[out_rows])
  pltpu.sync_copy(rv_v, rv_out.at[out_rows])
  pltpu.sync_copy(bsg_v, bsg_out.at[out_rows])
  pltpu.sync_copy(bog_v, bog_out.at[out_rows])


@jax.jit
def _sc_gather(order3, r_idx3, u_hi3, v_hi3, u_lo3, v_lo3,
               uv2, rvh, Wu, bs16, bo16):
  nw, n_chunks, _ = r_idx3.shape
  uv_chunks = order3.shape[1]
  bpw = n_chunks * IDX_CHUNK
  upw = uv_chunks * IDX_CHUNK
  b = nw * bpw
  dim = uv2.shape[1]
  f32 = jnp.float32
  i32 = jnp.int32
  mesh = plsc.VectorSubcoreMesh(core_axis_name="c", subcore_axis_name="s")
  idx_t = pltpu.VMEM((n_chunks, IDX_CHUNK), i32)
  run = pl.kernel(
      _gather_body,
      mesh=mesh,
      compiler_params=pltpu.CompilerParams(use_tc_tiling_on_sc=False,
                                           needs_layout_passes=False),
      out_type=[
          jax.ShapeDtypeStruct((2 * b, dim), f32),
          jax.ShapeDtypeStruct((b, dim), f32),
          jax.ShapeDtypeStruct((b, dim), f32),
          jax.ShapeDtypeStruct((b,), f32),
          jax.ShapeDtypeStruct((b,), f32),
      ],
      scratch_types=[
          pltpu.VMEM((uv_chunks, IDX_CHUNK), i32),
          idx_t, idx_t, idx_t, idx_t, idx_t,
          pltpu.VMEM((upw, dim), f32),
          pltpu.VMEM((bpw, dim), f32),
          pltpu.VMEM((bpw, dim), f32),
          pltpu.VMEM((bpw, LANES), f32),
          pltpu.VMEM((bpw, LANES), f32),
          pltpu.VMEM((bpw,), f32),
          pltpu.VMEM((bpw,), f32),
          pltpu.SemaphoreType.DMA,
      ],
  )
  return run(order3, r_idx3, u_hi3, v_hi3, u_lo3, v_lo3,
             uv2, rvh, Wu, bs16, bo16)


def _artanh(x):
  return 0.5 * jnp.log((1 + x) / (1 - x))


def _sqnorm(x):
  return jnp.sum(x * x, axis=0, keepdims=True)


def _norm(x):
  return jnp.sqrt(_sqnorm(x))


def _proj(t, eps=1e-5):
  nrm = _norm(t)
  msk = (nrm >= 1).astype(t.dtype)
  return t / (nrm - eps) * msk + t * (1 - msk)


def _p_sum(x, y):
  sqxnorm = jnp.clip(_sqnorm(x), 0.0, 1 - 1e-5)
  sqynorm = jnp.clip(_sqnorm(y), 0.0, 1 - 1e-5)
  dotxy = jnp.sum(x * y, axis=0, keepdims=True)
  numerator = (1 + 2 * dotxy + sqynorm) * x + (1 - sqxnorm) * y
  denominator = 1 + 2 * dotxy + sqxnorm * sqynorm
  return numerator / denominator


def _math_body(u_ref, v_ref, ru_ref, rv_ref, bs_ref, bo_ref, out_ref):
  u = _proj(u_ref[...].T)
  v = _proj(v_ref[...].T)
  rvh_g = _proj(rv_ref[...].T)
  Ru = ru_ref[...].T

  normu = jnp.clip(_norm(u), 1e-10, 1 - 1e-5)
  u_e = _artanh(normu) * u / normu
  u_W = u_e * Ru
  normw = jnp.clip(_norm(u_W), 1e-10, None)
  u_m = jnp.tanh(normw) * u_W / normw
  v_m = _p_sum(v, rvh_g)
  u_m = _proj(u_m)
  v_m = _proj(v_m)
  d = _p_sum(-u_m, v_m)
  nrm = jnp.clip(jnp.sqrt(jnp.sum(d * d, axis=0)), 1e-10, 1 - 1e-5)
  sqdist = (2.0 * _artanh(nrm)) ** 2
  out_ref[...] = -sqdist + bs_ref[...][:, 0] + bo_ref[...][:, 0]


@jax.jit
def _tc_math(u, v, ru, rv, bsg, bog):
  b, dim = u.shape
  blk = 2048
  grid = (b // blk,)
  row_spec = pl.BlockSpec((blk, dim), lambda i: (i, 0))
  one_spec = pl.BlockSpec((blk, 1), lambda i: (i, 0))
  return pl.pallas_call(
      _math_body,
      grid=grid,
      in_specs=[row_spec, row_spec, row_spec, row_spec, one_spec, one_spec],
      out_specs=pl.BlockSpec((blk,), lambda i: (i,)),
      out_shape=jax.ShapeDtypeStruct((b,), jnp.float32),
  )(u, v, ru, rv, bsg, bog)


def kernel(u_idx, r_idx, v_idx, Eh, rvh, Wu, bs, bo):
  b = u_idx.shape[0]
  n_chunks = b // (NW * IDX_CHUNK)
  shape3 = (NW, n_chunks, IDX_CHUNK)
  u_idx = u_idx.astype(jnp.int32)
  r_idx = r_idx.astype(jnp.int32)
  v_idx = v_idx.astype(jnp.int32)

  ent = jnp.concatenate([u_idx, v_idx])
  ents_sorted, order = jax.lax.sort(
      (ent, jnp.arange(2 * b, dtype=jnp.int32)), num_keys=1, is_stable=False)
  order3 = order.reshape(NW, 2 * n_chunks, IDX_CHUNK)
  ents2 = ents_sorted.reshape(NW, (2 * b) // NW)

  EhT = jnp.swapaxes(Eh, 0, 1)
  uv_sorted = _sc_stream_gather(EhT, ents2)[0]
  uv2 = jnp.swapaxes(uv_sorted, 0, 1)

  r_idx3 = r_idx.reshape(shape3)
  u_hi3 = (u_idx >> 4).reshape(shape3)
  v_hi3 = (v_idx >> 4).reshape(shape3)
  u_lo3 = (u_idx & (LANES - 1)).reshape(shape3)
  v_lo3 = (v_idx & (LANES - 1)).reshape(shape3)
  bs16 = bs.reshape(-1, LANES)
  bo16 = bo.reshape(-1, LANES)
  uv_unperm, ru, rv, bsg, bog = _sc_gather(order3, r_idx3,
                                           u_hi3, v_hi3, u_lo3, v_lo3,
                                           uv2, rvh, Wu, bs16, bo16)
  u = uv_unperm[:b]
  v = uv_unperm[b:]
  return _tc_math(u, v, ru, rv, bsg[:, None], bog[:, None])


# speedup vs baseline: 1.7305x; 1.0023x over previous
import functools

import jax
import jax.numpy as jnp
from jax import lax
from jax.experimental import pallas as pl
from jax.experimental.pallas import tpu as pltpu
from jax.experimental.pallas import tpu_sc as plsc

NC = 2
NS = 16
NW = NC * NS
IDX_CHUNK = 128
LANES = 16
WIN = 2944


def _stream_body(eht_hbm, ents_hbm, out_hbm, ent_v, win_v, soa_v):
  n_ent = ent_v.shape[0]
  dim, n_table = eht_hbm.shape
  pad_minor = ((n_table + 127) // 128) * 128
  cb_max = (pad_minor - WIN) // 128
  wid = lax.axis_index("s") * NC + lax.axis_index("c")
  pltpu.sync_copy(ents_hbm.at[wid], ent_v)

  iota = lax.iota(jnp.int32, LANES)
  zeros = jnp.zeros((LANES,), jnp.int32)

  def group(g, cb):
    evec = ent_v[pl.ds(g * LANES, LANES)]
    for j in range(LANES):
      e = evec[j]
      trig = (e - cb * 128) >= WIN
      newcb = jnp.minimum(lax.shift_right_logical(e, 7), cb_max)
      cb = jnp.where(trig, newcb, cb)

      @pl.when(trig)
      def _():
        off = pl.multiple_of(cb * 128, 128)
        pltpu.sync_copy(eht_hbm.at[:, pl.ds(off, WIN)], win_v)

      col = zeros + (e - cb * 128)
      k = zeros + (g * LANES + j)
      lo = plsc.load_gather(win_v, [iota, col])
      hi = plsc.load_gather(win_v, [iota + LANES, col])
      plsc.store_scatter(soa_v, [iota, k], lo)
      plsc.store_scatter(soa_v, [iota + LANES, k], hi)
    return cb

  lax.fori_loop(0, n_ent // LANES, group, jnp.int32(-(2 ** 20)))
  pltpu.sync_copy(soa_v, out_hbm.at[:, pl.ds(wid * n_ent, n_ent)])


@jax.jit
def _sc_stream_gather(EhT, ents2):
  nw, n_ent = ents2.shape
  dim = EhT.shape[0]
  mesh = plsc.VectorSubcoreMesh(core_axis_name="c", subcore_axis_name="s")
  run = pl.kernel(
      _stream_body,
      mesh=mesh,
      compiler_params=pltpu.CompilerParams(use_tc_tiling_on_sc=True,
                                           needs_layout_passes=False,
                                           disable_bounds_checks=True),
      out_type=[jax.ShapeDtypeStruct((dim, nw * n_ent), jnp.float32)],
      scratch_types=[
          pltpu.VMEM((n_ent,), jnp.int32),
          pltpu.VMEM((dim, WIN), jnp.float32),
          pltpu.VMEM((dim, n_ent), jnp.float32),
      ],
  )
  return run(EhT, ents2)


def _gather_body(order_hbm, ridx_hbm, uhi_hbm, vhi_hbm,
                 ulo_hbm, vlo_hbm, uv_hbm, rvh_hbm, wu_hbm,
                 bs_hbm, bo_hbm,
                 uv_out, ru_out, rv_out, bsg_out, bog_out,
                 order_v, ridx_v, uhi_v, vhi_v, ulo_v, vlo_v,
                 uv_rows, ru_v, rv_v, bsr_v, bor_v, bsg_v, bog_v, sem):
  n_chunks = ridx_v.shape[0]
  bpw = n_chunks * IDX_CHUNK
  uv_chunks = order_v.shape[0]
  upw = uv_chunks * IDX_CHUNK
  wid = lax.axis_index("s") * NC + lax.axis_index("c")
  base = wid * bpw

  pltpu.sync_copy(order_hbm.at[wid], order_v)
  pltpu.sync_copy(ridx_hbm.at[wid], ridx_v)
  pltpu.sync_copy(uhi_hbm.at[wid], uhi_v)
  pltpu.sync_copy(vhi_hbm.at[wid], vhi_v)
  pltpu.sync_copy(ulo_hbm.at[wid], ulo_v)
  pltpu.sync_copy(vlo_hbm.at[wid], vlo_v)
  pltpu.sync_copy(uv_hbm.at[pl.ds(wid * upw, upw)], uv_rows)

  copies = []
  for c in range(uv_chunks):
    rows = pl.ds(c * IDX_CHUNK, IDX_CHUNK)
    copies.append(pltpu.async_copy(uv_rows.at[rows],
                                   uv_out.at[order_v.at[c]], sem))
  for c in range(n_chunks):
    rows = pl.ds(c * IDX_CHUNK, IDX_CHUNK)
    copies.append(pltpu.async_copy(wu_hbm.at[ridx_v.at[c]], ru_v.at[rows], sem))
    copies.append(pltpu.async_copy(rvh_hbm.at[ridx_v.at[c]], rv_v.at[rows], sem))
    copies.append(pltpu.async_copy(bs_hbm.at[uhi_v.at[c]], bsr_v.at[rows], sem))
    copies.append(pltpu.async_copy(bo_hbm.at[vhi_v.at[c]], bor_v.at[rows], sem))
  for cp in copies:
    cp.wait()

  for g in range(bpw // LANES):
    row_ids = g * LANES + lax.iota(jnp.int32, LANES)
    c = (g * LANES) // IDX_CHUNK
    o = (g * LANES) % IDX_CHUNK
    ucol = ulo_v[c, pl.ds(o, LANES)]
    vcol = vlo_v[c, pl.ds(o, LANES)]
    bsg_v[pl.ds(g * LANES, LANES)] = plsc.load_gather(bsr_v, [row_ids, ucol])
    bog_v[pl.ds(g * LANES, LANES)] = plsc.load_gather(bor_v, [row_ids, vcol])

  out_rows = pl.ds(base, bpw)
  pltpu.sync_copy(ru_v, ru_out.at[out_rows])
  pltpu.sync_copy(rv_v, rv_out.at[out_rows])
  pltpu.sync_copy(bsg_v, bsg_out.at[out_rows])
  pltpu.sync_copy(bog_v, bog_out.at[out_rows])


@jax.jit
def _sc_gather(order3, r_idx3, u_hi3, v_hi3, u_lo3, v_lo3,
               uv2, rvh, Wu, bs16, bo16):
  nw, n_chunks, _ = r_idx3.shape
  uv_chunks = order3.shape[1]
  bpw = n_chunks * IDX_CHUNK
  upw = uv_chunks * IDX_CHUNK
  b = nw * bpw
  dim = uv2.shape[1]
  f32 = jnp.float32
  i32 = jnp.int32
  mesh = plsc.VectorSubcoreMesh(core_axis_name="c", subcore_axis_name="s")
  idx_t = pltpu.VMEM((n_chunks, IDX_CHUNK), i32)
  run = pl.kernel(
      _gather_body,
      mesh=mesh,
      compiler_params=pltpu.CompilerParams(use_tc_tiling_on_sc=False,
                                           needs_layout_passes=False),
      out_type=[
          jax.ShapeDtypeStruct((2 * b, dim), f32),
          jax.ShapeDtypeStruct((b, dim), f32),
          jax.ShapeDtypeStruct((b, dim), f32),
          jax.ShapeDtypeStruct((b,), f32),
          jax.ShapeDtypeStruct((b,), f32),
      ],
      scratch_types=[
          pltpu.VMEM((uv_chunks, IDX_CHUNK), i32),
          idx_t, idx_t, idx_t, idx_t, idx_t,
          pltpu.VMEM((upw, dim), f32),
          pltpu.VMEM((bpw, dim), f32),
          pltpu.VMEM((bpw, dim), f32),
          pltpu.VMEM((bpw, LANES), f32),
          pltpu.VMEM((bpw, LANES), f32),
          pltpu.VMEM((bpw,), f32),
          pltpu.VMEM((bpw,), f32),
          pltpu.SemaphoreType.DMA,
      ],
  )
  return run(order3, r_idx3, u_hi3, v_hi3, u_lo3, v_lo3,
             uv2, rvh, Wu, bs16, bo16)


def _artanh(x):
  return 0.5 * jnp.log((1 + x) / (1 - x))


def _sqnorm(x):
  return jnp.sum(x * x, axis=0, keepdims=True)


def _norm(x):
  return jnp.sqrt(_sqnorm(x))


def _proj(t, eps=1e-5):
  nrm = _norm(t)
  msk = (nrm >= 1).astype(t.dtype)
  return t / (nrm - eps) * msk + t * (1 - msk)


def _p_sum(x, y):
  sqxnorm = jnp.clip(_sqnorm(x), 0.0, 1 - 1e-5)
  sqynorm = jnp.clip(_sqnorm(y), 0.0, 1 - 1e-5)
  dotxy = jnp.sum(x * y, axis=0, keepdims=True)
  numerator = (1 + 2 * dotxy + sqynorm) * x + (1 - sqxnorm) * y
  denominator = 1 + 2 * dotxy + sqxnorm * sqynorm
  return numerator / denominator


def _math_body(u_ref, v_ref, ru_ref, rv_ref, bs_ref, bo_ref, out_ref):
  u = _proj(u_ref[...].T)
  v = _proj(v_ref[...].T)
  rvh_g = _proj(rv_ref[...].T)
  Ru = ru_ref[...].T

  normu = jnp.clip(_norm(u), 1e-10, 1 - 1e-5)
  u_e = _artanh(normu) * u / normu
  u_W = u_e * Ru
  normw = jnp.clip(_norm(u_W), 1e-10, None)
  u_m = jnp.tanh(normw) * u_W / normw
  v_m = _p_sum(v, rvh_g)
  u_m = _proj(u_m)
  v_m = _proj(v_m)
  d = _p_sum(-u_m, v_m)
  nrm = jnp.clip(jnp.sqrt(jnp.sum(d * d, axis=0)), 1e-10, 1 - 1e-5)
  sqdist = (2.0 * _artanh(nrm)) ** 2
  out_ref[...] = -sqdist + bs_ref[...][:, 0] + bo_ref[...][:, 0]


@jax.jit
def _tc_math(u, v, ru, rv, bsg, bog):
  b, dim = u.shape
  blk = 2048
  grid = (b // blk,)
  row_spec = pl.BlockSpec((blk, dim), lambda i: (i, 0))
  one_spec = pl.BlockSpec((blk, 1), lambda i: (i, 0))
  return pl.pallas_call(
      _math_body,
      grid=grid,
      in_specs=[row_spec, row_spec, row_spec, row_spec, one_spec, one_spec],
      out_specs=pl.BlockSpec((blk,), lambda i: (i,)),
      out_shape=jax.ShapeDtypeStruct((b,), jnp.float32),
  )(u, v, ru, rv, bsg, bog)


def kernel(u_idx, r_idx, v_idx, Eh, rvh, Wu, bs, bo):
  b = u_idx.shape[0]
  n_chunks = b // (NW * IDX_CHUNK)
  shape3 = (NW, n_chunks, IDX_CHUNK)
  u_idx = u_idx.astype(jnp.int32)
  r_idx = r_idx.astype(jnp.int32)
  v_idx = v_idx.astype(jnp.int32)

  ent = jnp.concatenate([u_idx, v_idx])
  ents_sorted, order = jax.lax.sort(
      (ent, jnp.arange(2 * b, dtype=jnp.int32)), num_keys=1, is_stable=False)
  order3 = order.reshape(NW, 2 * n_chunks, IDX_CHUNK)
  ents2 = ents_sorted.reshape(NW, (2 * b) // NW)

  EhT = jnp.swapaxes(Eh, 0, 1)
  uv_sorted = _sc_stream_gather(EhT, ents2)[0]
  uv2 = jnp.swapaxes(uv_sorted, 0, 1)

  r_idx3 = r_idx.reshape(shape3)
  u_hi3 = (u_idx >> 4).reshape(shape3)
  v_hi3 = (v_idx >> 4).reshape(shape3)
  u_lo3 = (u_idx & (LANES - 1)).reshape(shape3)
  v_lo3 = (v_idx & (LANES - 1)).reshape(shape3)
  bs16 = bs.reshape(-1, LANES)
  bo16 = bo.reshape(-1, LANES)
  uv_unperm, ru, rv, bsg, bog = _sc_gather(order3, r_idx3,
                                           u_hi3, v_hi3, u_lo3, v_lo3,
                                           uv2, rvh, Wu, bs16, bo16)
  u = uv_unperm[:b]
  v = uv_unperm[b:]
  return _tc_math(u, v, ru, rv, bsg[:, None], bog[:, None])
